# Initial kernel scaffold; baseline (speedup 1.0000x reference)
import jax
import jax.numpy as jnp
from jax.experimental import pallas as pl


def kernel(latent, codebook):
    raise NotImplementedError("write your pallas kernel here")


# speedup vs baseline: 1.2837x; 1.2837x over previous
import functools

import jax
import jax.numpy as jnp
from jax import lax
from jax.experimental import pallas as pl

_K = 128
_D = 32
_L = 8
_B = 16384
_BB = 256
_BF = _BB * _L


def _body(x_ref, cb_ref, cbflat_ref, q_ref, pol_ref, set_ref):
    x = x_ref[...]
    cb = cb_ref[...]
    c2 = jnp.sum(cb ** 2, axis=-1)
    x2 = jnp.sum(x ** 2, axis=-1, keepdims=True)
    m = lax.dot_general(x, cb, (((1,), (1,)), ((), ())),
                        preferred_element_type=jnp.float32)
    d = (x2 + c2[None, :]) - 2.0 * m
    dmin = jnp.min(d, axis=1, keepdims=True)
    iota = lax.broadcasted_iota(jnp.int32, (_BF, _K), 1)
    idx = jnp.min(jnp.where(d <= dmin, iota, _K), axis=1, keepdims=True)
    onehot = (iota == idx).astype(jnp.float32)
    q = lax.dot_general(onehot, cb, (((1,), (0,)), ((), ())),
                        preferred_element_type=jnp.float32)
    q_ref[...] = q
    pol_ref[...] = x + (q - x)
    set_ref[...] = jnp.broadcast_to(cbflat_ref[...], (_BB, _K * _D))


@jax.jit
def kernel(latent, codebook):
    xflat = latent.reshape(_B * _L, _D)
    cbflat = codebook.reshape(1, _K * _D)
    grid = _B // _BB
    q, pol, cbset = pl.pallas_call(
        _body,
        grid=(grid,),
        in_specs=[
            pl.BlockSpec((_BF, _D), lambda i: (i, 0)),
            pl.BlockSpec((_K, _D), lambda i: (0, 0)),
            pl.BlockSpec((1, _K * _D), lambda i: (0, 0)),
        ],
        out_specs=[
            pl.BlockSpec((_BF, _D), lambda i: (i, 0)),
            pl.BlockSpec((_BF, _D), lambda i: (i, 0)),
            pl.BlockSpec((_BB, _K * _D), lambda i: (i, 0)),
        ],
        out_shape=[
            jax.ShapeDtypeStruct((_B * _L, _D), jnp.float32),
            jax.ShapeDtypeStruct((_B * _L, _D), jnp.float32),
            jax.ShapeDtypeStruct((_B, _K * _D), jnp.float32),
        ],
    )(xflat, codebook, cbflat)
    return (pol.reshape(_B, _L * _D),
            q.reshape(_B, _L * _D),
            cbset.reshape(_B, _K, _D))
